# baseline (speedup 1.0000x reference)
import functools
import jax
import jax.numpy as jnp
from jax import lax
from jax.experimental import pallas as pl
from jax.experimental.pallas import tpu as pltpu
from jax.experimental.pallas import tpu_sc as plsc

_D = 32
_B2 = 32768
_NC = 2
_NS = 16
_NW = _NC * _NS
_BPW = _B2 // _NW

_mesh = plsc.VectorSubcoreMesh(core_axis_name="c", subcore_axis_name="s")


@functools.partial(
    pl.kernel,
    mesh=_mesh,
    out_type=jax.ShapeDtypeStruct((_B2, _D), jnp.float32),
    scratch_types=[
        pltpu.VMEM((_BPW,), jnp.int32),
        pltpu.VMEM((_BPW, _D), jnp.float32),
        pltpu.SemaphoreType.DMA,
    ],
    compiler_params=pltpu.CompilerParams(use_tc_tiling_on_sc=False),
)
def _gather_rows(table_hbm, idx_hbm, out_hbm, idx_v, rows_v, sem):
    wid = lax.axis_index("s") * _NC + lax.axis_index("c")
    base = wid * _BPW
    pltpu.sync_copy(idx_hbm.at[pl.ds(base, _BPW)], idx_v)
    pltpu.async_copy(table_hbm.at[idx_v], rows_v, sem).wait()
    pltpu.sync_copy(rows_v, out_hbm.at[pl.ds(base, _BPW)])


def kernel(action_indices, embedding_table):
    idx_flat = action_indices.astype(jnp.int32).reshape(-1)
    out = _gather_rows(embedding_table, idx_flat)
    return out.reshape(action_indices.shape[0], 2 * embedding_table.shape[1])


# speedup vs baseline: 1.0044x; 1.0044x over previous
import functools
import jax
import jax.numpy as jnp
from jax import lax
from jax.experimental import pallas as pl
from jax.experimental.pallas import tpu as pltpu
from jax.experimental.pallas import tpu_sc as plsc

_D = 32
_DP = 128
_B = 16384
_NC = 2
_NS = 16
_NW = _NC * _NS
_BPW = _B // _NW

_mesh = plsc.VectorSubcoreMesh(core_axis_name="c", subcore_axis_name="s")


@functools.partial(
    pl.kernel,
    mesh=_mesh,
    out_type=jax.ShapeDtypeStruct((_B, 2 * _D), jnp.float32),
    scratch_types=[
        pltpu.VMEM((_BPW,), jnp.int32),
        pltpu.VMEM((_BPW,), jnp.int32),
        pltpu.VMEM((_BPW // 2, _DP), jnp.float32),
        pltpu.VMEM((_BPW, 2 * _D), jnp.float32),
        pltpu.SemaphoreType.DMA,
    ],
)
def _gather_rows(table_hbm, idx_hbm, out_hbm, idx0_v, idx1_v, rows_v, cmp_v, sem):
    wid = lax.axis_index("s") * _NC + lax.axis_index("c")
    base = wid * _BPW
    half = _BPW // 2
    pltpu.sync_copy(idx_hbm.at[0, pl.ds(base, _BPW)], idx0_v)
    pltpu.sync_copy(idx_hbm.at[1, pl.ds(base, _BPW)], idx1_v)

    def compact(row_off, dst_off):
        def body(r, carry):
            cmp_v[row_off + r, pl.ds(dst_off, 16)] = rows_v[r, pl.ds(0, 16)]
            cmp_v[row_off + r, pl.ds(dst_off + 16, 16)] = rows_v[r, pl.ds(16, 16)]
            return carry

        lax.fori_loop(0, half, body, 0, unroll=8)

    for a, idx_v in ((0, idx0_v), (1, idx1_v)):
        for c in (0, 1):
            pltpu.async_copy(
                table_hbm.at[idx_v.at[pl.ds(c * half, half)]], rows_v, sem
            ).wait()
            compact(c * half, a * _D)
    pltpu.sync_copy(cmp_v, out_hbm.at[pl.ds(base, _BPW), :])


def kernel(action_indices, embedding_table):
    table_p = jnp.pad(embedding_table, ((0, 0), (0, _DP - _D)))
    idx_t = action_indices.astype(jnp.int32).T
    return _gather_rows(table_p, idx_t)
